# baseline (speedup 1.0000x reference)
import numpy as np

import jax
import jax.numpy as jnp
from jax import lax
from jax.experimental import pallas as pl
from jax.experimental.pallas import tpu as pltpu
from jax.experimental.pallas import tpu_sc as plsc

N = 10000
E = 320000
D = 128
U = 128

NC = 2
NS = 16
NW = NC * NS
C = 80
CH = 126
SUPER = 7
NSUP = CH // SUPER
EP = NW * CH * C
NP = 10240
N_PER_TILE = NP // NS

_k = np.arange(D)
_PERM = 32 * (_k // 32) + 2 * (_k % 16) + (_k % 32) // 16


def _sc_agg_body(nf_hbm, src_hbm, dst_hbm, w_hbm, out_hbm,
                 acc, src_v, dst_v, w_v, gat0, gat1, sc0, sc1,
                 g0, g1, s0, s1):
    gat = (gat0, gat1)
    scd = (sc0, sc1)
    gsems = (g0, g1)
    ssems = (s0, s1)
    cc = lax.axis_index("c")
    ss = lax.axis_index("s")
    tile = ss * NC + cc

    def zrow(i, carry):
        for cg in range(D // 16):
            sc0[i, pl.ds(cg * 16, 16)] = jnp.zeros((16,), jnp.float32)
        return carry

    lax.fori_loop(0, C, zrow, 0, unroll=False)

    def zcopy(k, carry):
        pltpu.sync_copy(sc0, acc.at[pl.ds(ss * N_PER_TILE + k * C, C)])
        return carry

    lax.fori_loop(0, N_PER_TILE // C, zcopy, 0, unroll=False)

    pltpu.sync_copy(src_hbm.at[tile, 0], src_v.at[0])

    plsc.subcore_barrier()

    for b in range(2):
        pltpu.async_copy(nf_hbm.at[src_v.at[0, b]], gat[b], gsems[b])

    def outer(o, carry):
        for b in range(2):
            j = 2 * o + b
            ksup = j // SUPER
            r = j - ksup * SUPER
            kb = lax.rem(ksup, 2)

            @pl.when(r == 0)
            def _():
                pltpu.sync_copy(dst_hbm.at[tile, ksup], dst_v.at[kb])
                pltpu.sync_copy(w_hbm.at[tile, ksup], w_v)

            pltpu.make_async_copy(nf_hbm.at[src_v.at[kb, r]], gat[b],
                                  gsems[b]).wait()

            @pl.when(o >= 1)
            def _():
                pltpu.make_async_copy(scd[b], acc.at[dst_v.at[kb, r]],
                                      ssems[b]).wait()

            def group_body(g, carry2):
                wvec = w_v[r, pl.ds(g * 16, 16)]
                for i in range(16):
                    wsc = wvec[i]
                    e = g * 16 + i
                    for q in range(D // 32):
                        ab = gat[b][e, pl.ds(q * 16, 16)]
                        lo = plsc.bitcast(ab << 16, jnp.float32)
                        hi = plsc.bitcast(ab & jnp.int32(-65536), jnp.float32)
                        scd[b][e, pl.ds(q * 32, 16)] = lo * wsc
                        scd[b][e, pl.ds(q * 32 + 16, 16)] = hi * wsc
                return carry2

            lax.fori_loop(0, C // 16, group_body, 0, unroll=False)

            pltpu.async_copy(scd[b], acc.at[dst_v.at[kb, r]], ssems[b],
                             add=True)

            jn = j + 2
            ksn = jn // SUPER
            rn = jn - ksn * SUPER
            kbn = lax.rem(ksn, 2)

            @pl.when((rn == 0) & (j <= CH - 3))
            def _():
                pltpu.sync_copy(src_hbm.at[tile, ksn], src_v.at[kbn])

            @pl.when(o <= CH // 2 - 2)
            def _():
                pltpu.async_copy(nf_hbm.at[src_v.at[kbn, rn]], gat[b],
                                 gsems[b])
        return carry

    lax.fori_loop(0, CH // 2, outer, 0, unroll=False)

    for b in range(2):
        pltpu.make_async_copy(scd[b], acc.at[dst_v.at[0, 0]],
                              ssems[b]).wait()

    plsc.subcore_barrier()

    def ocopy(k, carry):
        pltpu.sync_copy(acc.at[pl.ds(ss * N_PER_TILE + k * C, C)],
                        out_hbm.at[cc, pl.ds(ss * N_PER_TILE + k * C, C)])
        return carry

    lax.fori_loop(0, N_PER_TILE // C, ocopy, 0, unroll=False)


@jax.jit
def _sc_agg(nf_bf16, src4d, dst4d, w4d):
    return pl.kernel(
        _sc_agg_body,
        out_type=jax.ShapeDtypeStruct((NC, NP, D), jnp.float32),
        mesh=plsc.VectorSubcoreMesh(core_axis_name="c", subcore_axis_name="s"),
        compiler_params=pltpu.CompilerParams(use_tc_tiling_on_sc=False, needs_layout_passes=False),
        scratch_types=[
            pltpu.VMEM_SHARED((NP, D), jnp.float32),
            pltpu.VMEM((2, SUPER, C), jnp.int32),
            pltpu.VMEM((2, SUPER, C), jnp.int32),
            pltpu.VMEM((SUPER, C), jnp.float32),
            pltpu.VMEM((C, D // 2), jnp.int32),
            pltpu.VMEM((C, D // 2), jnp.int32),
            pltpu.VMEM((C, D), jnp.float32),
            pltpu.VMEM((C, D), jnp.float32),
            pltpu.SemaphoreType.DMA,
            pltpu.SemaphoreType.DMA,
            pltpu.SemaphoreType.DMA,
            pltpu.SemaphoreType.DMA,
        ],
    )(nf_bf16, src4d, dst4d, w4d)


def _tc_proj_body(p_ref, w_ref, b_ref, o_ref):
    a = p_ref[0] + p_ref[1]
    acc = jnp.dot(a, w_ref[...], preferred_element_type=jnp.float32)
    o_ref[...] = jnp.maximum(acc + b_ref[...], 0.0)


@jax.jit
def _tc_proj(partials, W, b2d):
    bm = 1000
    return pl.pallas_call(
        _tc_proj_body,
        grid=(N // bm,),
        in_specs=[
            pl.BlockSpec((NC, bm, D), lambda i: (0, i, 0)),
            pl.BlockSpec((D, U), lambda i: (0, 0)),
            pl.BlockSpec((1, U), lambda i: (0, 0)),
        ],
        out_specs=pl.BlockSpec((bm, U), lambda i: (i, 0)),
        out_shape=jax.ShapeDtypeStruct((N, U), jnp.float32),
    )(partials, W, b2d)


def kernel(node_features, edge_index, edge_weight, W, b):
    pad = EP - E
    src4d = jnp.concatenate(
        [edge_index[1], jnp.zeros((pad,), jnp.int32)]).reshape(NW, NSUP, SUPER, C)
    dst4d = jnp.concatenate(
        [edge_index[0], jnp.zeros((pad,), jnp.int32)]).reshape(NW, NSUP, SUPER, C)
    w4d = jnp.concatenate(
        [edge_weight, jnp.zeros((pad,), jnp.float32)]).reshape(NW, NSUP, SUPER, C)
    nf_pk = lax.bitcast_convert_type(
        node_features.astype(jnp.bfloat16).reshape(N, D // 2, 2),
        jnp.int32)
    partials = _sc_agg(nf_pk, src4d, dst4d, w4d)
    w_perm = W[_PERM, :]
    return _tc_proj(partials, w_perm, b.reshape(1, U))


# speedup vs baseline: 1.1785x; 1.1785x over previous
import jax
import jax.numpy as jnp
from jax import lax
from jax.experimental import pallas as pl
from jax.experimental.pallas import tpu as pltpu
from jax.experimental.pallas import tpu_sc as plsc

N = 10000
E = 320000
D = 128
U = 128

NC = 2
NS = 16
NW = NC * NS
C = 80
CH = 126
NBUF = 3
SUPER = 14
NSUP = CH // SUPER
EP = NW * CH * C
NP = 10240
N_PER_TILE = NP // NS


def _sc_agg_body(nf_hbm, src_hbm, dst_hbm, w_hbm, out_hbm,
                 acc, src_v, dst_v, w_v, rows_v,
                 g0, g1, g2, s0, s1, s2):
    gsems = (g0, g1, g2)
    ssems = (s0, s1, s2)
    cc = lax.axis_index("c")
    ss = lax.axis_index("s")
    tile = ss * NC + cc

    def zrow(i, carry):
        for cg in range(D // 16):
            rows_v[NBUF - 1, i, pl.ds(cg * 16, 16)] = jnp.zeros((16,), jnp.float32)
        return carry

    lax.fori_loop(0, C, zrow, 0, unroll=False)

    def zcopy(k, carry):
        pltpu.sync_copy(rows_v.at[NBUF - 1],
                        acc.at[pl.ds(ss * N_PER_TILE + k * C, C)])
        return carry

    lax.fori_loop(0, N_PER_TILE // C, zcopy, 0, unroll=False)

    pltpu.sync_copy(src_hbm.at[tile, 0], src_v.at[0])

    plsc.subcore_barrier()

    for b in range(NBUF):
        pltpu.async_copy(nf_hbm.at[src_v.at[0, b]], rows_v.at[b], gsems[b])

    def outer(o, carry):
        for b in range(NBUF):
            j = NBUF * o + b
            ksup = j // SUPER
            r = j - ksup * SUPER
            kb = lax.rem(ksup, 2)

            @pl.when(r == 0)
            def _():
                pltpu.sync_copy(dst_hbm.at[tile, ksup], dst_v.at[kb])
                pltpu.sync_copy(w_hbm.at[tile, ksup], w_v)

            pltpu.make_async_copy(nf_hbm.at[src_v.at[kb, r]], rows_v.at[b],
                                  gsems[b]).wait()

            def group_body(g, carry2):
                wvec = w_v[r, pl.ds(g * 16, 16)]
                for i in range(16):
                    wsc = wvec[i]
                    e = g * 16 + i
                    for cg in range(D // 16):
                        sl = pl.ds(cg * 16, 16)
                        rows_v[b, e, sl] = rows_v[b, e, sl] * wsc
                return carry2

            lax.fori_loop(0, C // 16, group_body, 0, unroll=False)

            pltpu.async_copy(rows_v.at[b], acc.at[dst_v.at[kb, r]], ssems[b],
                             add=True)

            jn = j + 2
            ksn = jn // SUPER
            rn = jn - ksn * SUPER
            kbn = lax.rem(ksn, 2)

            @pl.when((rn == 0) & (j <= CH - 3))
            def _():
                pltpu.sync_copy(src_hbm.at[tile, ksn], src_v.at[kbn])

            bn = (b + 2) % NBUF
            pred = (o >= 1) if b == 0 else (o <= CH // NBUF - 2)

            @pl.when(pred)
            def _():
                pltpu.make_async_copy(rows_v.at[bn], acc.at[dst_v.at[kb, r]],
                                      ssems[bn]).wait()
                pltpu.async_copy(nf_hbm.at[src_v.at[kbn, rn]], rows_v.at[bn],
                                 gsems[bn])
        return carry

    lax.fori_loop(0, CH // NBUF, outer, 0, unroll=False)

    for b in range(NBUF):
        pltpu.make_async_copy(rows_v.at[b], acc.at[dst_v.at[0, 0]],
                              ssems[b]).wait()

    plsc.subcore_barrier()

    def ocopy(k, carry):
        pltpu.sync_copy(acc.at[pl.ds(ss * N_PER_TILE + k * C, C)],
                        out_hbm.at[cc, pl.ds(ss * N_PER_TILE + k * C, C)])
        return carry

    lax.fori_loop(0, N_PER_TILE // C, ocopy, 0, unroll=False)


@jax.jit
def _sc_agg(nf, src4d, dst4d, w4d):
    return pl.kernel(
        _sc_agg_body,
        out_type=jax.ShapeDtypeStruct((NC, NP, D), jnp.float32),
        mesh=plsc.VectorSubcoreMesh(core_axis_name="c", subcore_axis_name="s"),
        scratch_types=[
            pltpu.VMEM_SHARED((NP, D), jnp.float32),
            pltpu.VMEM((2, SUPER, C), jnp.int32),
            pltpu.VMEM((2, SUPER, C), jnp.int32),
            pltpu.VMEM((SUPER, C), jnp.float32),
            pltpu.VMEM((NBUF, C, D), jnp.float32),
            pltpu.SemaphoreType.DMA,
            pltpu.SemaphoreType.DMA,
            pltpu.SemaphoreType.DMA,
            pltpu.SemaphoreType.DMA,
            pltpu.SemaphoreType.DMA,
            pltpu.SemaphoreType.DMA,
        ],
    )(nf, src4d, dst4d, w4d)


def _tc_proj_body(p_ref, w_ref, b_ref, o_ref):
    a = p_ref[0] + p_ref[1]
    acc = jnp.dot(a, w_ref[...], preferred_element_type=jnp.float32)
    o_ref[...] = jnp.maximum(acc + b_ref[...], 0.0)


@jax.jit
def _tc_proj(partials, W, b2d):
    bm = 1000
    return pl.pallas_call(
        _tc_proj_body,
        grid=(N // bm,),
        in_specs=[
            pl.BlockSpec((NC, bm, D), lambda i: (0, i, 0)),
            pl.BlockSpec((D, U), lambda i: (0, 0)),
            pl.BlockSpec((1, U), lambda i: (0, 0)),
        ],
        out_specs=pl.BlockSpec((bm, U), lambda i: (i, 0)),
        out_shape=jax.ShapeDtypeStruct((N, U), jnp.float32),
    )(partials, W, b2d)


def kernel(node_features, edge_index, edge_weight, W, b):
    pad = EP - E
    src4d = jnp.concatenate(
        [edge_index[1], jnp.zeros((pad,), jnp.int32)]).reshape(NW, NSUP, SUPER, C)
    dst4d = jnp.concatenate(
        [edge_index[0], jnp.zeros((pad,), jnp.int32)]).reshape(NW, NSUP, SUPER, C)
    w4d = jnp.concatenate(
        [edge_weight, jnp.zeros((pad,), jnp.float32)]).reshape(NW, NSUP, SUPER, C)
    partials = _sc_agg(node_features, src4d, dst4d, w4d)
    return _tc_proj(partials, W, b.reshape(1, U))


# speedup vs baseline: 1.1854x; 1.0059x over previous
import jax
import jax.numpy as jnp
from jax import lax
from jax.experimental import pallas as pl
from jax.experimental.pallas import tpu as pltpu
from jax.experimental.pallas import tpu_sc as plsc

N = 10000
E = 320000
D = 128
U = 128

NC = 2
NS = 16
NW = NC * NS
C = 96
CH = 105
NBUF = 3
SUPER = 15
NSUP = CH // SUPER
EP = NW * CH * C
NP = 10240
N_PER_TILE = NP // NS
CCOPY = 80


def _sc_agg_body(nf_hbm, src_hbm, dst_hbm, w_hbm, out_hbm,
                 acc, src_v, dst_v, w_v, rows_v,
                 g0, g1, g2, s0, s1, s2):
    gsems = (g0, g1, g2)
    ssems = (s0, s1, s2)
    cc = lax.axis_index("c")
    ss = lax.axis_index("s")
    tile = ss * NC + cc

    def zrow(i, carry):
        for cg in range(D // 16):
            rows_v[NBUF - 1, i, pl.ds(cg * 16, 16)] = jnp.zeros((16,), jnp.float32)
        return carry

    lax.fori_loop(0, C, zrow, 0, unroll=False)

    def zcopy(k, carry):
        pltpu.sync_copy(rows_v.at[NBUF - 1, pl.ds(0, CCOPY)],
                        acc.at[pl.ds(ss * N_PER_TILE + k * CCOPY, CCOPY)])
        return carry

    lax.fori_loop(0, N_PER_TILE // CCOPY, zcopy, 0, unroll=False)

    pltpu.sync_copy(src_hbm.at[tile, 0], src_v.at[0])

    plsc.subcore_barrier()

    for b in range(NBUF):
        pltpu.async_copy(nf_hbm.at[src_v.at[0, b]], rows_v.at[b], gsems[b])

    def outer(o, carry):
        for b in range(NBUF):
            j = NBUF * o + b
            ksup = j // SUPER
            r = j - ksup * SUPER
            kb = lax.rem(ksup, 2)

            @pl.when(r == 0)
            def _():
                pltpu.sync_copy(dst_hbm.at[tile, ksup], dst_v.at[kb])
                pltpu.sync_copy(w_hbm.at[tile, ksup], w_v)

            pltpu.make_async_copy(nf_hbm.at[src_v.at[kb, r]], rows_v.at[b],
                                  gsems[b]).wait()

            def group_body(g, carry2):
                wvec = w_v[r, pl.ds(g * 16, 16)]
                for i in range(16):
                    wsc = wvec[i]
                    e = g * 16 + i
                    for cg in range(D // 16):
                        sl = pl.ds(cg * 16, 16)
                        rows_v[b, e, sl] = rows_v[b, e, sl] * wsc
                return carry2

            lax.fori_loop(0, C // 16, group_body, 0, unroll=False)

            pltpu.async_copy(rows_v.at[b], acc.at[dst_v.at[kb, r]], ssems[b],
                             add=True)

            jn = j + 2
            ksn = jn // SUPER
            rn = jn - ksn * SUPER
            kbn = lax.rem(ksn, 2)

            @pl.when((rn == 0) & (j <= CH - 3))
            def _():
                pltpu.sync_copy(src_hbm.at[tile, ksn], src_v.at[kbn])

            bn = (b + 2) % NBUF
            pred = (o >= 1) if b == 0 else (o <= CH // NBUF - 2)

            @pl.when(pred)
            def _():
                pltpu.make_async_copy(rows_v.at[bn], acc.at[dst_v.at[kb, r]],
                                      ssems[bn]).wait()
                pltpu.async_copy(nf_hbm.at[src_v.at[kbn, rn]], rows_v.at[bn],
                                 gsems[bn])
        return carry

    lax.fori_loop(0, CH // NBUF, outer, 0, unroll=False)

    for b in range(NBUF):
        pltpu.make_async_copy(rows_v.at[b], acc.at[dst_v.at[0, 0]],
                              ssems[b]).wait()

    plsc.subcore_barrier()

    def ocopy(k, carry):
        pltpu.sync_copy(acc.at[pl.ds(ss * N_PER_TILE + k * CCOPY, CCOPY)],
                        out_hbm.at[cc, pl.ds(ss * N_PER_TILE + k * CCOPY, CCOPY)])
        return carry

    lax.fori_loop(0, N_PER_TILE // CCOPY, ocopy, 0, unroll=False)


@jax.jit
def _sc_agg(nf, src4d, dst4d, w4d):
    return pl.kernel(
        _sc_agg_body,
        out_type=jax.ShapeDtypeStruct((NC, NP, D), jnp.float32),
        mesh=plsc.VectorSubcoreMesh(core_axis_name="c", subcore_axis_name="s"),
        scratch_types=[
            pltpu.VMEM_SHARED((NP, D), jnp.float32),
            pltpu.VMEM((2, SUPER, C), jnp.int32),
            pltpu.VMEM((2, SUPER, C), jnp.int32),
            pltpu.VMEM((SUPER, C), jnp.float32),
            pltpu.VMEM((NBUF, C, D), jnp.float32),
            pltpu.SemaphoreType.DMA,
            pltpu.SemaphoreType.DMA,
            pltpu.SemaphoreType.DMA,
            pltpu.SemaphoreType.DMA,
            pltpu.SemaphoreType.DMA,
            pltpu.SemaphoreType.DMA,
        ],
    )(nf, src4d, dst4d, w4d)


def _tc_proj_body(p_ref, w_ref, b_ref, o_ref):
    a = p_ref[0] + p_ref[1]
    acc = jnp.dot(a, w_ref[...], preferred_element_type=jnp.float32)
    o_ref[...] = jnp.maximum(acc + b_ref[...], 0.0)


@jax.jit
def _tc_proj(partials, W, b2d):
    bm = 1000
    return pl.pallas_call(
        _tc_proj_body,
        grid=(N // bm,),
        in_specs=[
            pl.BlockSpec((NC, bm, D), lambda i: (0, i, 0)),
            pl.BlockSpec((D, U), lambda i: (0, 0)),
            pl.BlockSpec((1, U), lambda i: (0, 0)),
        ],
        out_specs=pl.BlockSpec((bm, U), lambda i: (i, 0)),
        out_shape=jax.ShapeDtypeStruct((N, U), jnp.float32),
    )(partials, W, b2d)


def kernel(node_features, edge_index, edge_weight, W, b):
    pad = EP - E
    src4d = jnp.concatenate(
        [edge_index[1], jnp.zeros((pad,), jnp.int32)]).reshape(NW, NSUP, SUPER, C)
    dst4d = jnp.concatenate(
        [edge_index[0], jnp.zeros((pad,), jnp.int32)]).reshape(NW, NSUP, SUPER, C)
    w4d = jnp.concatenate(
        [edge_weight, jnp.zeros((pad,), jnp.float32)]).reshape(NW, NSUP, SUPER, C)
    partials = _sc_agg(node_features, src4d, dst4d, w4d)
    return _tc_proj(partials, W, b.reshape(1, U))
